# baseline (speedup 1.0000x reference)
import jax
import jax.numpy as jnp
from jax.experimental import pallas as pl
from jax.experimental.pallas import tpu as pltpu

D = 4096
H = 32
HD = 128
KV_LEN = 4095
PAGE = 16
TOPK = 64
NP = 256
TB = 16
TBS = 256
NPB = TBS // PAGE


def _mv(w_ref, x):
    wb = w_ref[...].astype(jnp.bfloat16).astype(jnp.float32)
    xb = x.astype(jnp.bfloat16).astype(jnp.float32)
    return jnp.sum(wb * xb, axis=1, keepdims=True)


def _qkv_body(x_ref, wq_ref, wk_ref, wv_ref, cos_ref, sin_ref,
              q_ref, k_ref, v_ref):
    x = x_ref[...]
    cos = cos_ref[...]
    sin = sin_ref[...]

    def rope(u):
        rot = jnp.concatenate([-u[HD // 2:, :], u[:HD // 2, :]], axis=0)
        return u * cos + rot * sin

    q_ref[...] = rope(_mv(wq_ref, x))
    k_ref[...] = rope(_mv(wk_ref, x))
    v_ref[...] = _mv(wv_ref, x)


def _scan_body(k_ref, knew_ref, q_ref, logit_ref, est_ref):
    tb = pl.program_id(1)
    kb = k_ref[...]
    kn = knew_ref[0]
    q = q_ref[0]
    row = jax.lax.broadcasted_iota(jnp.int32, (TBS, HD), 0)
    tok = tb * TBS + row
    kb = jnp.where(tok == KV_LEN, kn, kb)
    scale = 1.0 / jnp.sqrt(jnp.float32(HD))
    lcol = jnp.sum(kb * q, axis=1, keepdims=True) * scale
    logit_ref[...] = lcol.reshape(1, TBS, 1)
    kp = kb.reshape(NPB, PAGE, HD)
    pmax = kp.max(axis=1)
    pmin = kp.min(axis=1)
    m = jnp.maximum(q * pmax, q * pmin)
    est_ref[...] = jnp.sum(m, axis=1, keepdims=True).reshape(1, NPB, 1)


def _topk_body(est_ref, sel_ref):
    est = est_ref[...]
    lane = jax.lax.broadcasted_iota(jnp.int32, (H, NP), 1)

    def step(_, carry):
        work, mask = carry
        m = jnp.max(work, axis=1, keepdims=True)
        first = jnp.min(jnp.where(work == m, lane, NP), axis=1, keepdims=True)
        hit = lane == first
        return (jnp.where(hit, -jnp.inf, work),
                jnp.maximum(mask, hit.astype(jnp.float32)))

    _, mask = jax.lax.fori_loop(
        0, TOPK, step, (est, jnp.zeros((H, NP), jnp.float32)))
    sel_ref[...] = mask


def _attend_body(logit_ref, sel_ref, v_ref, vnew_ref, out_ref,
                 acc_ref, m_ref, s_ref):
    tb = pl.program_id(1)

    @pl.when(tb == 0)
    def _init():
        m_ref[...] = jnp.full((1, 1), -1e30, jnp.float32)
        s_ref[...] = jnp.zeros((1, 1), jnp.float32)
        acc_ref[...] = jnp.zeros((1, HD), jnp.float32)

    l = logit_ref[0]
    selp = sel_ref[0]
    mask_t = jnp.broadcast_to(
        selp.reshape(NPB, 1, 1), (NPB, PAGE, 1)).reshape(TBS, 1)

    vb = v_ref[...]
    vn = vnew_ref[0]
    row = jax.lax.broadcasted_iota(jnp.int32, (TBS, HD), 0)
    tok = tb * TBS + row
    vb = jnp.where(tok == KV_LEN, vn, vb)

    lm = jnp.where(mask_t > 0.5, l, -1e30)
    local_max = jnp.max(lm, axis=0, keepdims=True)
    prev_m = m_ref[...]
    m_new = jnp.maximum(prev_m, local_max)
    p = jnp.exp(lm - m_new) * mask_t
    corr = jnp.exp(prev_m - m_new)
    s_new = s_ref[...] * corr + jnp.sum(p, axis=0, keepdims=True)
    pv = jnp.sum(p * vb, axis=0, keepdims=True)
    acc_new = acc_ref[...] * corr + pv
    acc_ref[...] = acc_new
    m_ref[...] = m_new
    s_ref[...] = s_new

    @pl.when(tb == TB - 1)
    def _fin():
        out_ref[0] = acc_new / s_new


def _oproj_body(ctx_ref, wo_ref, out_ref):
    out_ref[...] = _mv(wo_ref, ctx_ref[...])


def kernel(hidden_states, position_ids, k_cache, v_cache, Wq, Wk, Wv, Wo):
    f32 = jnp.float32
    x = hidden_states.reshape(1, D).astype(f32)
    pos = position_ids[0, 0].astype(f32)
    half = HD // 2
    inv_freq = 1.0 / (10000.0 ** (jnp.arange(0, half, dtype=f32) / half))
    ang = pos * inv_freq
    cos = jnp.concatenate([jnp.cos(ang), jnp.cos(ang)])
    sin = jnp.concatenate([jnp.sin(ang), jnp.sin(ang)])
    cos_col = jnp.tile(cos, H).reshape(D, 1)
    sin_col = jnp.tile(sin, H).reshape(D, 1)

    q_col, knew_col, vnew_col = pl.pallas_call(
        _qkv_body,
        grid=(H,),
        in_specs=[
            pl.BlockSpec((1, D), lambda h: (0, 0)),
            pl.BlockSpec((HD, D), lambda h: (h, 0)),
            pl.BlockSpec((HD, D), lambda h: (h, 0)),
            pl.BlockSpec((HD, D), lambda h: (h, 0)),
            pl.BlockSpec((HD, 1), lambda h: (h, 0)),
            pl.BlockSpec((HD, 1), lambda h: (h, 0)),
        ],
        out_specs=[
            pl.BlockSpec((HD, 1), lambda h: (h, 0)),
            pl.BlockSpec((HD, 1), lambda h: (h, 0)),
            pl.BlockSpec((HD, 1), lambda h: (h, 0)),
        ],
        out_shape=[jax.ShapeDtypeStruct((D, 1), f32)] * 3,
    )(x, Wq, Wk, Wv, cos_col, sin_col)
    q3 = q_col.reshape(H, 1, HD)
    knew3 = knew_col.reshape(H, 1, HD)
    vnew3 = vnew_col.reshape(H, 1, HD)

    logits, est = pl.pallas_call(
        _scan_body,
        grid=(H, TB),
        in_specs=[
            pl.BlockSpec((TBS, HD), lambda h, tb: (tb, h)),
            pl.BlockSpec((1, 1, HD), lambda h, tb: (h, 0, 0)),
            pl.BlockSpec((1, 1, HD), lambda h, tb: (h, 0, 0)),
        ],
        out_specs=[
            pl.BlockSpec((1, TBS, 1), lambda h, tb: (h, tb, 0)),
            pl.BlockSpec((1, NPB, 1), lambda h, tb: (h, tb, 0)),
        ],
        out_shape=[
            jax.ShapeDtypeStruct((H, TB * TBS, 1), f32),
            jax.ShapeDtypeStruct((H, NP, 1), f32),
        ],
    )(k_cache.reshape(KV_LEN, D), knew3, q3)

    sel = pl.pallas_call(
        _topk_body,
        out_shape=jax.ShapeDtypeStruct((H, NP), f32),
    )(est.reshape(H, NP))

    ctx = pl.pallas_call(
        _attend_body,
        grid=(H, TB),
        in_specs=[
            pl.BlockSpec((1, TBS, 1), lambda h, tb: (h, tb, 0)),
            pl.BlockSpec((1, NPB, 1), lambda h, tb: (h, tb, 0)),
            pl.BlockSpec((TBS, HD), lambda h, tb: (tb, h)),
            pl.BlockSpec((1, 1, HD), lambda h, tb: (h, 0, 0)),
        ],
        out_specs=pl.BlockSpec((1, 1, HD), lambda h, tb: (h, 0, 0)),
        out_shape=jax.ShapeDtypeStruct((H, 1, HD), f32),
        scratch_shapes=[
            pltpu.VMEM((1, HD), f32),
            pltpu.VMEM((1, 1), f32),
            pltpu.VMEM((1, 1), f32),
        ],
    )(logits, sel.reshape(H, NP, 1), v_cache.reshape(KV_LEN, D), vnew3)

    ctx_row = ctx.reshape(1, D)
    out = pl.pallas_call(
        _oproj_body,
        grid=(H,),
        in_specs=[
            pl.BlockSpec((1, D), lambda h: (0, 0)),
            pl.BlockSpec((HD, D), lambda h: (h, 0)),
        ],
        out_specs=pl.BlockSpec((HD, 1), lambda h: (h, 0)),
        out_shape=jax.ShapeDtypeStruct((D, 1), f32),
    )(ctx_row, Wo)
    return out.reshape(1, 1, D)


# speedup vs baseline: 5.0697x; 5.0697x over previous
import jax
import jax.numpy as jnp
from jax.experimental import pallas as pl
from jax.experimental.pallas import tpu as pltpu

D = 4096
H = 32
HD = 128
KV_LEN = 4095
PAGE = 16
TOPK = 64
NP = 256
TB = 16
TBS = 256
NPB = TBS // PAGE


def _mv(w_ref, x):
    wb = w_ref[...].astype(jnp.bfloat16).astype(jnp.float32)
    xb = x.astype(jnp.bfloat16).astype(jnp.float32)
    return jnp.sum(wb * xb, axis=1, keepdims=True)


def _qkv_body(x_ref, wq_ref, wk_ref, wv_ref, cos_ref, sin_ref,
              q_ref, k_ref, v_ref):
    x = x_ref[...]
    cos = cos_ref[...]
    sin = sin_ref[...]

    def rope(u):
        rot = jnp.concatenate([-u[HD // 2:, :], u[:HD // 2, :]], axis=0)
        return u * cos + rot * sin

    q_ref[...] = rope(_mv(wq_ref, x))
    k_ref[...] = rope(_mv(wk_ref, x))
    v_ref[...] = _mv(wv_ref, x)


def _scan_body(k_ref, knew_ref, q_ref, logit_ref, est_ref):
    tb = pl.program_id(0)
    kb = k_ref[...]
    kn = knew_ref[...]
    q = q_ref[...]
    row = jax.lax.broadcasted_iota(jnp.int32, (TBS, H, HD), 0)
    tok = tb * TBS + row
    kb = jnp.where(tok == KV_LEN, kn, kb)
    scale = 1.0 / jnp.sqrt(jnp.float32(HD))
    logit_ref[...] = jnp.sum(kb * q, axis=2) * scale
    kp = kb.reshape(NPB, PAGE, H, HD)
    pmax = kp.max(axis=1)
    pmin = kp.min(axis=1)
    m = jnp.maximum(q * pmax, q * pmin)
    est_ref[...] = jnp.sum(m, axis=2)


def _topk_body(est_ref, sel_ref):
    est = est_ref[...]
    riota = jax.lax.broadcasted_iota(jnp.int32, (NP, H), 0)

    def step(_, carry):
        work, mask = carry
        m = jnp.max(work, axis=0, keepdims=True)
        first = jnp.min(jnp.where(work == m, riota, NP), axis=0, keepdims=True)
        hit = riota == first
        return (jnp.where(hit, -jnp.inf, work),
                jnp.maximum(mask, hit.astype(jnp.float32)))

    _, mask = jax.lax.fori_loop(
        0, TOPK, step, (est, jnp.zeros((NP, H), jnp.float32)))
    sel_ref[...] = mask


def _attend_body(logit_ref, sel_ref, v_ref, vnew_ref, out_ref,
                 acc_ref, m_ref, s_ref):
    tb = pl.program_id(0)

    @pl.when(tb == 0)
    def _init():
        m_ref[...] = jnp.full((1, H), -1e30, jnp.float32)
        s_ref[...] = jnp.zeros((1, H), jnp.float32)
        acc_ref[...] = jnp.zeros((H, HD), jnp.float32)

    l = logit_ref[...]
    selp = sel_ref[...]
    mask_t = jnp.broadcast_to(
        selp.reshape(NPB, 1, H), (NPB, PAGE, H)).reshape(TBS, H)

    vb = v_ref[...]
    vn = vnew_ref[...]
    row = jax.lax.broadcasted_iota(jnp.int32, (TBS, H, HD), 0)
    tok = tb * TBS + row
    vb = jnp.where(tok == KV_LEN, vn, vb)

    lm = jnp.where(mask_t > 0.5, l, -1e30)
    local_max = jnp.max(lm, axis=0, keepdims=True)
    prev_m = m_ref[...]
    m_new = jnp.maximum(prev_m, local_max)
    p = jnp.exp(lm - m_new) * mask_t
    corr = jnp.exp(prev_m - m_new)
    s_new = s_ref[...] * corr + jnp.sum(p, axis=0, keepdims=True)
    pv = jnp.sum(p.reshape(TBS, H, 1) * vb, axis=0)
    acc_new = acc_ref[...] * corr.reshape(H, 1) + pv

    acc_ref[...] = acc_new
    m_ref[...] = m_new
    s_ref[...] = s_new

    @pl.when(tb == TB - 1)
    def _fin():
        sb = jnp.broadcast_to(s_new, (H, H))
        ri = jax.lax.broadcasted_iota(jnp.int32, (H, H), 0)
        ci = jax.lax.broadcasted_iota(jnp.int32, (H, H), 1)
        s_col = jnp.sum(jnp.where(ri == ci, sb, 0.0), axis=1, keepdims=True)
        out_ref[...] = acc_new / s_col


def _oproj_body(ctx_ref, wo_ref, out_ref):
    out_ref[...] = _mv(wo_ref, ctx_ref[...])


def kernel(hidden_states, position_ids, k_cache, v_cache, Wq, Wk, Wv, Wo):
    f32 = jnp.float32
    x = hidden_states.reshape(1, D).astype(f32)
    pos = position_ids[0, 0].astype(f32)
    half = HD // 2
    inv_freq = 1.0 / (10000.0 ** (jnp.arange(0, half, dtype=f32) / half))
    ang = pos * inv_freq
    cos = jnp.concatenate([jnp.cos(ang), jnp.cos(ang)])
    sin = jnp.concatenate([jnp.sin(ang), jnp.sin(ang)])
    cos_col = jnp.tile(cos, H).reshape(D, 1)
    sin_col = jnp.tile(sin, H).reshape(D, 1)

    q_col, knew_col, vnew_col = pl.pallas_call(
        _qkv_body,
        grid=(H,),
        in_specs=[
            pl.BlockSpec((1, D), lambda h: (0, 0)),
            pl.BlockSpec((HD, D), lambda h: (h, 0)),
            pl.BlockSpec((HD, D), lambda h: (h, 0)),
            pl.BlockSpec((HD, D), lambda h: (h, 0)),
            pl.BlockSpec((HD, 1), lambda h: (h, 0)),
            pl.BlockSpec((HD, 1), lambda h: (h, 0)),
        ],
        out_specs=[
            pl.BlockSpec((HD, 1), lambda h: (h, 0)),
            pl.BlockSpec((HD, 1), lambda h: (h, 0)),
            pl.BlockSpec((HD, 1), lambda h: (h, 0)),
        ],
        out_shape=[jax.ShapeDtypeStruct((D, 1), f32)] * 3,
    )(x, Wq, Wk, Wv, cos_col, sin_col)
    q3 = q_col.reshape(1, H, HD)
    knew3 = knew_col.reshape(1, H, HD)
    vnew3 = vnew_col.reshape(1, H, HD)

    logits, est = pl.pallas_call(
        _scan_body,
        grid=(TB,),
        in_specs=[
            pl.BlockSpec((TBS, H, HD), lambda tb: (tb, 0, 0)),
            pl.BlockSpec((1, H, HD), lambda tb: (0, 0, 0)),
            pl.BlockSpec((1, H, HD), lambda tb: (0, 0, 0)),
        ],
        out_specs=[
            pl.BlockSpec((TBS, H), lambda tb: (tb, 0)),
            pl.BlockSpec((NPB, H), lambda tb: (tb, 0)),
        ],
        out_shape=[
            jax.ShapeDtypeStruct((TB * TBS, H), f32),
            jax.ShapeDtypeStruct((NP, H), f32),
        ],
    )(k_cache, knew3, q3)

    sel = pl.pallas_call(
        _topk_body,
        out_shape=jax.ShapeDtypeStruct((NP, H), f32),
    )(est)

    ctx = pl.pallas_call(
        _attend_body,
        grid=(TB,),
        in_specs=[
            pl.BlockSpec((TBS, H), lambda tb: (tb, 0)),
            pl.BlockSpec((NPB, H), lambda tb: (tb, 0)),
            pl.BlockSpec((TBS, H, HD), lambda tb: (tb, 0, 0)),
            pl.BlockSpec((1, H, HD), lambda tb: (0, 0, 0)),
        ],
        out_specs=pl.BlockSpec((H, HD), lambda tb: (0, 0)),
        out_shape=jax.ShapeDtypeStruct((H, HD), f32),
        scratch_shapes=[
            pltpu.VMEM((H, HD), f32),
            pltpu.VMEM((1, H), f32),
            pltpu.VMEM((1, H), f32),
        ],
    )(logits, sel, v_cache, vnew3)

    ctx_row = ctx.reshape(1, D)
    out = pl.pallas_call(
        _oproj_body,
        grid=(H,),
        in_specs=[
            pl.BlockSpec((1, D), lambda h: (0, 0)),
            pl.BlockSpec((HD, D), lambda h: (h, 0)),
        ],
        out_specs=pl.BlockSpec((HD, 1), lambda h: (h, 0)),
        out_shape=jax.ShapeDtypeStruct((D, 1), f32),
    )(ctx_row, Wo)
    return out.reshape(1, 1, D)


# speedup vs baseline: 5.4400x; 1.0730x over previous
import jax
import jax.numpy as jnp
from jax.experimental import pallas as pl
from jax.experimental.pallas import tpu as pltpu

D = 4096
H = 32
HD = 128
KV_LEN = 4095
PAGE = 16
TOPK = 64
NP = 256
TB = 16
TBS = 256
NPB = TBS // PAGE

_G_SCAN = H
_G_TOPK = _G_SCAN + TB
_G_ATT = _G_TOPK + 1
_G_END = _G_ATT + TB - 1


def _mv(w_ref, x):
    wb = w_ref[...].astype(jnp.bfloat16).astype(jnp.float32)
    xb = x.astype(jnp.bfloat16).astype(jnp.float32)
    return jnp.sum(wb * xb, axis=1, keepdims=True)


def _t_col_to_row(col, n):
    b = jnp.broadcast_to(col, (n, n))
    ri = jax.lax.broadcasted_iota(jnp.int32, (n, n), 0)
    ci = jax.lax.broadcasted_iota(jnp.int32, (n, n), 1)
    return jnp.sum(jnp.where(ri == ci, b, 0.0), axis=0, keepdims=True)


def _fused_body(x_ref, wq_ref, wk_ref, wv_ref, cos_ref, sin_ref,
                k_ref, v_ref, out_ref,
                q_scr, kn_scr, vn_scr, logit_scr, est_scr, sel_scr,
                acc_ref, m_ref, s_ref):
    g = pl.program_id(0)

    @pl.when(g < _G_SCAN)
    def _qkv():
        x = x_ref[...]
        cos = cos_ref[...]
        sin = sin_ref[...]

        def rope(u):
            rot = jnp.concatenate([-u[HD // 2:, :], u[:HD // 2, :]], axis=0)
            return u * cos + rot * sin

        h = g
        q_scr[pl.ds(h, 1), :] = _t_col_to_row(rope(_mv(wq_ref, x)), HD)
        kn_scr[pl.ds(h, 1), :] = _t_col_to_row(rope(_mv(wk_ref, x)), HD)
        vn_scr[pl.ds(h, 1), :] = _t_col_to_row(_mv(wv_ref, x), HD)

    @pl.when((g >= _G_SCAN) & (g < _G_TOPK))
    def _scan():
        tb = g - _G_SCAN
        kb = k_ref[...]
        kn = kn_scr[...].reshape(1, H, HD)
        q = q_scr[...].reshape(1, H, HD)
        row = jax.lax.broadcasted_iota(jnp.int32, (TBS, H, HD), 0)
        tok = tb * TBS + row
        kb = jnp.where(tok == KV_LEN, kn, kb)
        scale = 1.0 / jnp.sqrt(jnp.float32(HD))
        logit_scr[pl.ds(tb * TBS, TBS), :] = jnp.sum(kb * q, axis=2) * scale
        kp = kb.reshape(NPB, PAGE, H, HD)
        pmax = kp.max(axis=1)
        pmin = kp.min(axis=1)
        m = jnp.maximum(q * pmax, q * pmin)
        est_scr[pl.ds(tb * NPB, NPB), :] = jnp.sum(m, axis=2)

    @pl.when(g == _G_TOPK)
    def _topk():
        est = est_scr[...]
        riota = jax.lax.broadcasted_iota(jnp.int32, (NP, H), 0)

        def step(_, carry):
            work, mask = carry
            mm = jnp.max(work, axis=0, keepdims=True)
            first = jnp.min(jnp.where(work == mm, riota, NP),
                            axis=0, keepdims=True)
            hit = riota == first
            return (jnp.where(hit, -jnp.inf, work),
                    jnp.maximum(mask, hit.astype(jnp.float32)))

        _, mask = jax.lax.fori_loop(
            0, TOPK, step, (est, jnp.zeros((NP, H), jnp.float32)))
        sel_scr[...] = mask
        m_ref[...] = jnp.full((1, H), -1e30, jnp.float32)
        s_ref[...] = jnp.zeros((1, H), jnp.float32)
        acc_ref[...] = jnp.zeros((H, HD), jnp.float32)

    @pl.when(g >= _G_ATT)
    def _attend():
        tb = g - _G_ATT
        l = logit_scr[pl.ds(tb * TBS, TBS), :]
        selp = sel_scr[pl.ds(tb * NPB, NPB), :]
        mask_t = jnp.broadcast_to(
            selp.reshape(NPB, 1, H), (NPB, PAGE, H)).reshape(TBS, H)

        vb = v_ref[...]
        vn = vn_scr[...].reshape(1, H, HD)
        row = jax.lax.broadcasted_iota(jnp.int32, (TBS, H, HD), 0)
        tok = tb * TBS + row
        vb = jnp.where(tok == KV_LEN, vn, vb)

        lm = jnp.where(mask_t > 0.5, l, -1e30)
        local_max = jnp.max(lm, axis=0, keepdims=True)
        prev_m = m_ref[...]
        m_new = jnp.maximum(prev_m, local_max)
        p = jnp.exp(lm - m_new) * mask_t
        corr = jnp.exp(prev_m - m_new)
        s_new = s_ref[...] * corr + jnp.sum(p, axis=0, keepdims=True)
        pv = jnp.sum(p.reshape(TBS, H, 1) * vb, axis=0)
        acc_new = acc_ref[...] * corr.reshape(H, 1) + pv

        acc_ref[...] = acc_new
        m_ref[...] = m_new
        s_ref[...] = s_new

        @pl.when(g == _G_END)
        def _fin():
            sb = jnp.broadcast_to(s_new, (H, H))
            ri = jax.lax.broadcasted_iota(jnp.int32, (H, H), 0)
            ci = jax.lax.broadcasted_iota(jnp.int32, (H, H), 1)
            s_col = jnp.sum(jnp.where(ri == ci, sb, 0.0),
                            axis=1, keepdims=True)
            out_ref[...] = acc_new / s_col


def _oproj_body(ctx_ref, wo_ref, out_ref):
    out_ref[...] = _mv(wo_ref, ctx_ref[...])


def kernel(hidden_states, position_ids, k_cache, v_cache, Wq, Wk, Wv, Wo):
    f32 = jnp.float32
    x = hidden_states.reshape(1, D).astype(f32)
    pos = position_ids[0, 0].astype(f32)
    half = HD // 2
    inv_freq = 1.0 / (10000.0 ** (jnp.arange(0, half, dtype=f32) / half))
    ang = pos * inv_freq
    cos = jnp.concatenate([jnp.cos(ang), jnp.cos(ang)])
    sin = jnp.concatenate([jnp.sin(ang), jnp.sin(ang)])
    cos_col = jnp.tile(cos, H).reshape(D, 1)
    sin_col = jnp.tile(sin, H).reshape(D, 1)

    nsteps = _G_END + 1
    ctx = pl.pallas_call(
        _fused_body,
        grid=(nsteps,),
        in_specs=[
            pl.BlockSpec((1, D), lambda g: (0, 0)),
            pl.BlockSpec((HD, D), lambda g: (jnp.minimum(g, H - 1), 0)),
            pl.BlockSpec((HD, D), lambda g: (jnp.minimum(g, H - 1), 0)),
            pl.BlockSpec((HD, D), lambda g: (jnp.minimum(g, H - 1), 0)),
            pl.BlockSpec((HD, 1), lambda g: (jnp.minimum(g, H - 1), 0)),
            pl.BlockSpec((HD, 1), lambda g: (jnp.minimum(g, H - 1), 0)),
            pl.BlockSpec(
                (TBS, H, HD),
                lambda g: (jnp.clip(g - _G_SCAN, 0, TB - 1), 0, 0)),
            pl.BlockSpec(
                (TBS, H, HD),
                lambda g: (jnp.clip(g - _G_ATT, 0, TB - 1), 0, 0)),
        ],
        out_specs=pl.BlockSpec((H, HD), lambda g: (0, 0)),
        out_shape=jax.ShapeDtypeStruct((H, HD), f32),
        scratch_shapes=[
            pltpu.VMEM((H, HD), f32),
            pltpu.VMEM((H, HD), f32),
            pltpu.VMEM((H, HD), f32),
            pltpu.VMEM((TB * TBS, H), f32),
            pltpu.VMEM((NP, H), f32),
            pltpu.VMEM((NP, H), f32),
            pltpu.VMEM((H, HD), f32),
            pltpu.VMEM((1, H), f32),
            pltpu.VMEM((1, H), f32),
        ],
    )(x, Wq, Wk, Wv, cos_col, sin_col, k_cache, v_cache)

    ctx_row = ctx.reshape(1, D)
    out = pl.pallas_call(
        _oproj_body,
        grid=(H,),
        in_specs=[
            pl.BlockSpec((1, D), lambda h: (0, 0)),
            pl.BlockSpec((HD, D), lambda h: (h, 0)),
        ],
        out_specs=pl.BlockSpec((HD, 1), lambda h: (h, 0)),
        out_shape=jax.ShapeDtypeStruct((D, 1), f32),
    )(ctx_row, Wo)
    return out.reshape(1, 1, D)


# speedup vs baseline: 5.5038x; 1.0117x over previous
import jax
import jax.numpy as jnp
from jax.experimental import pallas as pl
from jax.experimental.pallas import tpu as pltpu

D = 4096
H = 32
HD = 128
KV_LEN = 4095
PAGE = 16
TOPK = 64
NP = 256
TB = 16
TBS = 256
NPB = TBS // PAGE

HPB = 2
QR = HPB * HD
_G_SCAN = H // HPB
_G_TOPK = _G_SCAN + TB
_G_ATT = _G_TOPK + 1
_G_END = _G_ATT + TB - 1


def _mv(w_ref, x):
    wb = w_ref[...].astype(jnp.bfloat16).astype(jnp.float32)
    xb = x.astype(jnp.bfloat16).astype(jnp.float32)
    return jnp.sum(wb * xb, axis=1, keepdims=True)


def _t_col_to_row(col, n):
    b = jnp.broadcast_to(col, (n, n))
    ri = jax.lax.broadcasted_iota(jnp.int32, (n, n), 0)
    ci = jax.lax.broadcasted_iota(jnp.int32, (n, n), 1)
    return jnp.sum(jnp.where(ri == ci, b, 0.0), axis=0, keepdims=True)


def _fused_body(x_ref, wq_ref, wk_ref, wv_ref, cos_ref, sin_ref,
                k_ref, v_ref, out_ref,
                q_scr, kn_scr, vn_scr, logit_scr, est_scr, sel_scr,
                acc_ref, m_ref, s_ref):
    g = pl.program_id(0)

    @pl.when(g < _G_SCAN)
    def _qkv():
        x = x_ref[...]
        cos = cos_ref[...]
        sin = sin_ref[...]

        def rope(u):
            u3 = u.reshape(HPB, HD, 1)
            rot = jnp.concatenate(
                [-u3[:, HD // 2:, :], u3[:, :HD // 2, :]], axis=1
            ).reshape(QR, 1)
            return u * cos + rot * sin

        def put(scr, col):
            c3 = col.reshape(HPB, HD, 1)
            for i in range(HPB):
                scr[pl.ds(g * HPB + i, 1), :] = _t_col_to_row(c3[i], HD)

        put(q_scr, rope(_mv(wq_ref, x)))
        put(kn_scr, rope(_mv(wk_ref, x)))
        put(vn_scr, _mv(wv_ref, x))

    @pl.when((g >= _G_SCAN) & (g < _G_TOPK))
    def _scan():
        tb = g - _G_SCAN
        kb = k_ref[...]
        kn = kn_scr[...].reshape(1, H, HD)
        q = q_scr[...].reshape(1, H, HD)
        row = jax.lax.broadcasted_iota(jnp.int32, (TBS, H, HD), 0)
        tok = tb * TBS + row
        kb = jnp.where(tok == KV_LEN, kn, kb)
        scale = 1.0 / jnp.sqrt(jnp.float32(HD))
        logit_scr[pl.ds(tb * TBS, TBS), :] = jnp.sum(kb * q, axis=2) * scale
        kp = kb.reshape(NPB, PAGE, H, HD)
        pmax = kp.max(axis=1)
        pmin = kp.min(axis=1)
        m = jnp.maximum(q * pmax, q * pmin)
        est_scr[pl.ds(tb * NPB, NPB), :] = jnp.sum(m, axis=2)

    @pl.when(g == _G_TOPK)
    def _topk():
        est = jnp.transpose(est_scr[...])
        liota = jax.lax.broadcasted_iota(jnp.int32, (H, NP), 1)

        def step(_, carry):
            work, mask = carry
            mm = jnp.max(work, axis=1, keepdims=True)
            first = jnp.min(jnp.where(work == mm, liota, NP),
                            axis=1, keepdims=True)
            hit = liota == first
            return (jnp.where(hit, -jnp.inf, work),
                    jnp.maximum(mask, hit.astype(jnp.float32)))

        _, mask = jax.lax.fori_loop(
            0, TOPK, step, (est, jnp.zeros((H, NP), jnp.float32)))
        sel_scr[...] = jnp.transpose(mask)
        m_ref[...] = jnp.full((1, H), -1e30, jnp.float32)
        s_ref[...] = jnp.zeros((1, H), jnp.float32)
        acc_ref[...] = jnp.zeros((H, HD), jnp.float32)

    @pl.when(g >= _G_ATT)
    def _attend():
        tb = g - _G_ATT
        l = logit_scr[pl.ds(tb * TBS, TBS), :]
        selp = sel_scr[pl.ds(tb * NPB, NPB), :]
        mask_t = jnp.broadcast_to(
            selp.reshape(NPB, 1, H), (NPB, PAGE, H)).reshape(TBS, H)

        vb = v_ref[...]
        vn = vn_scr[...].reshape(1, H, HD)
        row = jax.lax.broadcasted_iota(jnp.int32, (TBS, H, HD), 0)
        tok = tb * TBS + row
        vb = jnp.where(tok == KV_LEN, vn, vb)

        lm = jnp.where(mask_t > 0.5, l, -1e30)
        local_max = jnp.max(lm, axis=0, keepdims=True)
        prev_m = m_ref[...]
        m_new = jnp.maximum(prev_m, local_max)
        p = jnp.exp(lm - m_new) * mask_t
        corr = jnp.exp(prev_m - m_new)
        s_new = s_ref[...] * corr + jnp.sum(p, axis=0, keepdims=True)
        pv = jnp.sum(p.reshape(TBS, H, 1) * vb, axis=0)
        acc_new = acc_ref[...] * corr.reshape(H, 1) + pv

        acc_ref[...] = acc_new
        m_ref[...] = m_new
        s_ref[...] = s_new

        @pl.when(g == _G_END)
        def _fin():
            sb = jnp.broadcast_to(s_new, (H, H))
            ri = jax.lax.broadcasted_iota(jnp.int32, (H, H), 0)
            ci = jax.lax.broadcasted_iota(jnp.int32, (H, H), 1)
            s_col = jnp.sum(jnp.where(ri == ci, sb, 0.0),
                            axis=1, keepdims=True)
            out_ref[...] = acc_new / s_col


def _oproj_body(ctx_ref, wo_ref, out_ref):
    out_ref[...] = _mv(wo_ref, ctx_ref[...])


def kernel(hidden_states, position_ids, k_cache, v_cache, Wq, Wk, Wv, Wo):
    f32 = jnp.float32
    x = hidden_states.reshape(1, D).astype(f32)
    pos = position_ids[0, 0].astype(f32)
    half = HD // 2
    inv_freq = 1.0 / (10000.0 ** (jnp.arange(0, half, dtype=f32) / half))
    ang = pos * inv_freq
    cos = jnp.concatenate([jnp.cos(ang), jnp.cos(ang)])
    sin = jnp.concatenate([jnp.sin(ang), jnp.sin(ang)])
    cos_col = jnp.tile(cos, H).reshape(D, 1)
    sin_col = jnp.tile(sin, H).reshape(D, 1)

    nsteps = _G_END + 1
    ctx = pl.pallas_call(
        _fused_body,
        grid=(nsteps,),
        in_specs=[
            pl.BlockSpec((1, D), lambda g: (0, 0)),
            pl.BlockSpec((QR, D), lambda g: (jnp.minimum(g, _G_SCAN - 1), 0)),
            pl.BlockSpec((QR, D), lambda g: (jnp.minimum(g, _G_SCAN - 1), 0)),
            pl.BlockSpec((QR, D), lambda g: (jnp.minimum(g, _G_SCAN - 1), 0)),
            pl.BlockSpec((QR, 1), lambda g: (jnp.minimum(g, _G_SCAN - 1), 0)),
            pl.BlockSpec((QR, 1), lambda g: (jnp.minimum(g, _G_SCAN - 1), 0)),
            pl.BlockSpec(
                (TBS, H, HD),
                lambda g: (jnp.clip(g - _G_SCAN, 0, TB - 1), 0, 0)),
            pl.BlockSpec(
                (TBS, H, HD),
                lambda g: (jnp.clip(g - _G_ATT, 0, TB - 1), 0, 0)),
        ],
        out_specs=pl.BlockSpec((H, HD), lambda g: (0, 0)),
        out_shape=jax.ShapeDtypeStruct((H, HD), f32),
        scratch_shapes=[
            pltpu.VMEM((H, HD), f32),
            pltpu.VMEM((H, HD), f32),
            pltpu.VMEM((H, HD), f32),
            pltpu.VMEM((TB * TBS, H), f32),
            pltpu.VMEM((NP, H), f32),
            pltpu.VMEM((NP, H), f32),
            pltpu.VMEM((H, HD), f32),
            pltpu.VMEM((1, H), f32),
            pltpu.VMEM((1, H), f32),
        ],
    )(x, Wq, Wk, Wv, cos_col, sin_col, k_cache, v_cache)

    ctx_row = ctx.reshape(1, D)
    out = pl.pallas_call(
        _oproj_body,
        grid=(H // HPB,),
        in_specs=[
            pl.BlockSpec((1, D), lambda h: (0, 0)),
            pl.BlockSpec((QR, D), lambda h: (h, 0)),
        ],
        out_specs=pl.BlockSpec((QR, 1), lambda h: (h, 0)),
        out_shape=jax.ShapeDtypeStruct((D, 1), f32),
    )(ctx_row, Wo)
    return out.reshape(1, 1, D)
